# Initial kernel scaffold; baseline (speedup 1.0000x reference)
import functools

import jax
import jax.numpy as jnp
from jax import lax
from jax.experimental import pallas as pl
from jax.experimental.pallas import tpu as pltpu
from jax.experimental.pallas import tpu_sc as plsc

N = 10000
E = 320000
D = 128
DE = 16
H = 8
DH = 16

ACCW = 144
NW = 32
EPW = E // NW
C = 80
NCHUNK = EPW // C
RPS = N // 16


def _proj_body(x_ref, qw, qb, kw, kb, vw, vb, q_out, k_out, v_out):
    xb = x_ref[...]
    q_out[...] = jnp.dot(xb, qw[...], preferred_element_type=jnp.float32) + qb[...]
    k_out[...] = jnp.dot(xb, kw[...], preferred_element_type=jnp.float32) + kb[...]
    v_out[...] = jnp.dot(xb, vw[...], preferred_element_type=jnp.float32) + vb[...]


def _proj(x, qw, qb, kw, kb, vw, vb):
    blk = 1000
    grid = (N // blk,)
    w_spec = pl.BlockSpec((D, D), lambda i: (0, 0))
    b_spec = pl.BlockSpec((1, D), lambda i: (0, 0))
    r_spec = pl.BlockSpec((blk, D), lambda i: (i, 0))
    return pl.pallas_call(
        _proj_body,
        grid=grid,
        in_specs=[r_spec, w_spec, b_spec, w_spec, b_spec, w_spec, b_spec],
        out_specs=[r_spec, r_spec, r_spec],
        out_shape=[jax.ShapeDtypeStruct((N, D), jnp.float32)] * 3,
    )(x, qw, qb, kw, kb, vw, vb)


def _ee_body(ea_ref, w, b, out):
    out[...] = (jnp.dot(ea_ref[...], w[...], preferred_element_type=jnp.float32)
                + b[...]) * 0.25


def _ee(edge_attr, w, b):
    blk = 16000
    grid = (E // blk,)
    return pl.pallas_call(
        _ee_body,
        grid=grid,
        in_specs=[
            pl.BlockSpec((blk, DE), lambda i: (i, 0)),
            pl.BlockSpec((DE, D), lambda i: (0, 0)),
            pl.BlockSpec((1, D), lambda i: (0, 0)),
        ],
        out_specs=pl.BlockSpec((blk, D), lambda i: (i, 0)),
        out_shape=jax.ShapeDtypeStruct((E, D), jnp.float32),
    )(edge_attr, w, b)


def _edge_body(kh, qh, vh, eeh, srch, dsth, zrows, outh,
               srcv, dstv, kbuf, qbuf, vbuf, ebuf, msgbuf, acc, sem):
    cid = lax.axis_index("c")
    sid = lax.axis_index("s")
    wid = cid * 16 + sid
    base_e = wid * EPW
    r0 = sid * RPS

    pltpu.sync_copy(zrows, acc.at[pl.ds(r0, RPS)])

    zero16 = jnp.zeros((16,), jnp.float32)

    def _zrow(r, carry):
        msgbuf[r, pl.ds(D, 16)] = zero16
        return carry

    lax.fori_loop(0, C, _zrow, 0)

    plsc.subcore_barrier()

    iota16 = lax.iota(jnp.int32, 16)

    def _chunk(ci, carry):
        b = base_e + ci * C
        pltpu.sync_copy(srch.at[pl.ds(b, C)], srcv)
        pltpu.sync_copy(dsth.at[pl.ds(b, C)], dstv)
        cp_k = pltpu.async_copy(kh.at[srcv], kbuf, sem)
        cp_q = pltpu.async_copy(qh.at[dstv], qbuf, sem)
        cp_v = pltpu.async_copy(vh.at[srcv], vbuf, sem)
        cp_e = pltpu.async_copy(eeh.at[pl.ds(b, C)], ebuf, sem)
        cp_k.wait()
        cp_q.wait()
        cp_v.wait()
        cp_e.wait()
        for g in range(C // 16):
            rows = g * 16 + iota16
            for h in range(H):
                acc_v = None
                for dh in range(DH):
                    cols = jnp.full((16,), h * DH + dh, jnp.int32)
                    kv = plsc.load_gather(kbuf, [rows, cols])
                    qv = plsc.load_gather(qbuf, [rows, cols])
                    ev = plsc.load_gather(ebuf, [rows, cols])
                    t = kv * qv * ev
                    acc_v = t if acc_v is None else acc_v + t
                sc = jnp.exp(jnp.clip(acc_v, -5.0, 5.0))
                plsc.store_scatter(
                    msgbuf, [rows, jnp.full((16,), D + h, jnp.int32)], sc)
                for dh in range(DH):
                    cols = jnp.full((16,), h * DH + dh, jnp.int32)
                    vv = plsc.load_gather(vbuf, [rows, cols])
                    plsc.store_scatter(msgbuf, [rows, cols], vv * sc)
        pltpu.sync_copy(msgbuf, acc.at[dstv], add=True)
        return carry

    lax.fori_loop(0, NCHUNK, _chunk, 0)

    plsc.subcore_barrier()

    pltpu.sync_copy(acc.at[pl.ds(r0, RPS)], outh.at[cid, pl.ds(r0, RPS)])


def _edge(kh, qh, vh, ee, src, dst, zrows):
    mesh = plsc.VectorSubcoreMesh(core_axis_name="c", subcore_axis_name="s")
    return pl.kernel(
        _edge_body,
        out_type=jax.ShapeDtypeStruct((2, N, ACCW), jnp.float32),
        mesh=mesh,
        scratch_types=[
            pltpu.VMEM((C,), jnp.int32),
            pltpu.VMEM((C,), jnp.int32),
            pltpu.VMEM((C, D), jnp.float32),
            pltpu.VMEM((C, D), jnp.float32),
            pltpu.VMEM((C, D), jnp.float32),
            pltpu.VMEM((C, D), jnp.float32),
            pltpu.VMEM((C, ACCW), jnp.float32),
            pltpu.VMEM_SHARED((N, ACCW), jnp.float32),
            pltpu.SemaphoreType.DMA,
        ],
    )(kh, qh, vh, ee, src, dst, zrows)


def _final_body(p_ref, out):
    wv = p_ref[0, :, 0:D] + p_ref[1, :, 0:D]
    z = p_ref[0, :, D:D + H] + p_ref[1, :, D:D + H]
    blk = wv.shape[0]
    zb = jnp.broadcast_to(z.reshape(blk, H, 1), (blk, H, DH)).reshape(blk, D)
    out[...] = wv / (zb + 1e-6)


def _final(parts):
    blk = 1000
    grid = (N // blk,)
    return pl.pallas_call(
        _final_body,
        grid=grid,
        in_specs=[pl.BlockSpec((2, blk, ACCW), lambda i: (0, i, 0))],
        out_specs=pl.BlockSpec((blk, D), lambda i: (i, 0)),
        out_shape=jax.ShapeDtypeStruct((N, D), jnp.float32),
    )(parts)


def kernel(x, edge_index, edge_attr, batch_vec, Q_W, Q_b, K_W, K_b,
           E_W, E_b, V_W, V_b):
    qh, kh, vh = _proj(x, Q_W, Q_b.reshape(1, D), K_W, K_b.reshape(1, D),
                       V_W, V_b.reshape(1, D))
    ee = _ee(edge_attr, E_W, E_b.reshape(1, D))
    src = edge_index[0]
    dst = edge_index[1]
    zrows = jnp.zeros((RPS, ACCW), jnp.float32)
    parts = _edge(kh, qh, vh, ee, src, dst, zrows)
    return _final(parts)


# speedup vs baseline: 13.3519x; 13.3519x over previous
import jax
import jax.numpy as jnp
from jax import lax
from jax.experimental import pallas as pl
from jax.experimental.pallas import tpu as pltpu
from jax.experimental.pallas import tpu_sc as plsc

N = 10000
E = 320000
D = 128
DE = 16
H = 8
DH = 16

HD2 = 64
ACCW = 80
NPAD = 10240
NW = 32
EPW = E // NW
C = 80
NCHUNK = EPW // C
RPS = NPAD // 16


def _split(res):
    return jnp.stack([res[:, :HD2], res[:, HD2:]], axis=0)


def _proj_body(x_ref, qw, qb, kw, kb, vw, vb, q_out, k_out, v_out):
    xb = x_ref[...]
    q_out[...] = _split(
        jnp.dot(xb, qw[...], preferred_element_type=jnp.float32) + qb[...])
    k_out[...] = _split(
        jnp.dot(xb, kw[...], preferred_element_type=jnp.float32) + kb[...])
    v_out[...] = _split(
        jnp.dot(xb, vw[...], preferred_element_type=jnp.float32) + vb[...])


def _proj(x, qw, qb, kw, kb, vw, vb):
    blk = 1000
    grid = (N // blk,)
    w_spec = pl.BlockSpec((D, D), lambda i: (0, 0))
    b_spec = pl.BlockSpec((1, D), lambda i: (0, 0))
    x_spec = pl.BlockSpec((blk, D), lambda i: (i, 0))
    o_spec = pl.BlockSpec((2, blk, HD2), lambda i: (0, i, 0))
    return pl.pallas_call(
        _proj_body,
        grid=grid,
        in_specs=[x_spec, w_spec, b_spec, w_spec, b_spec, w_spec, b_spec],
        out_specs=[o_spec, o_spec, o_spec],
        out_shape=[jax.ShapeDtypeStruct((2, N, HD2), jnp.float32)] * 3,
    )(x, qw, qb, kw, kb, vw, vb)


def _ee_body(ea_ref, w, b, out):
    out[...] = _split(
        (jnp.dot(ea_ref[...], w[...], preferred_element_type=jnp.float32)
         + b[...]) * 0.25)


def _ee(edge_attr, w, b):
    blk = 16000
    grid = (E // blk,)
    return pl.pallas_call(
        _ee_body,
        grid=grid,
        in_specs=[
            pl.BlockSpec((blk, DE), lambda i: (i, 0)),
            pl.BlockSpec((DE, D), lambda i: (0, 0)),
            pl.BlockSpec((1, D), lambda i: (0, 0)),
        ],
        out_specs=pl.BlockSpec((2, blk, HD2), lambda i: (0, i, 0)),
        out_shape=jax.ShapeDtypeStruct((2, E, HD2), jnp.float32),
    )(edge_attr, w, b)


def _edge_body(kh, qh, vh, eeh, srch, dsth, zrows, outh,
               srcv, dstv, kbuf, qbuf, vbuf, ebuf, msgbuf, acc, sem):
    cid = lax.axis_index("c")
    sid = lax.axis_index("s")
    wid = cid * 16 + sid
    base_e = wid * EPW
    r0 = sid * RPS

    iota16 = lax.iota(jnp.int32, 16)

    for p in range(2):
        pltpu.sync_copy(zrows, acc.at[pl.ds(r0, RPS)])
        plsc.subcore_barrier()

        def _chunk(ci, carry):
            b = base_e + ci * C
            pltpu.sync_copy(srch.at[pl.ds(b, C)], srcv)
            pltpu.sync_copy(dsth.at[pl.ds(b, C)], dstv)
            cp_k = pltpu.async_copy(kh.at[p].at[srcv], kbuf, sem)
            cp_q = pltpu.async_copy(qh.at[p].at[dstv], qbuf, sem)
            cp_v = pltpu.async_copy(vh.at[p].at[srcv], vbuf, sem)
            cp_e = pltpu.async_copy(eeh.at[p].at[pl.ds(b, C)], ebuf, sem)
            cp_k.wait()
            cp_q.wait()
            cp_v.wait()
            cp_e.wait()

            def _edge_compute(e, carry2):
                svec = jnp.zeros((16,), jnp.float32)
                for h in range(H // 2):
                    ks = kbuf[e, pl.ds(h * DH, DH)]
                    qs = qbuf[e, pl.ds(h * DH, DH)]
                    es = ebuf[e, pl.ds(h * DH, DH)]
                    s = jnp.sum(ks * qs * es)
                    sv = jnp.exp(
                        jnp.clip(jnp.broadcast_to(s, (16,)), -5.0, 5.0))
                    vs = vbuf[e, pl.ds(h * DH, DH)]
                    msgbuf[e, pl.ds(h * DH, DH)] = vs * sv
                    svec = jnp.where(iota16 == h, sv, svec)
                msgbuf[e, pl.ds(HD2, 16)] = svec
                return carry2

            lax.fori_loop(0, C, _edge_compute, 0)
            pltpu.sync_copy(msgbuf, acc.at[dstv], add=True)
            return carry

        lax.fori_loop(0, NCHUNK, _chunk, 0)
        plsc.subcore_barrier()

        pltpu.sync_copy(acc.at[pl.ds(r0, RPS)],
                        outh.at[cid].at[p].at[pl.ds(r0, RPS)])


def _edge(kh, qh, vh, ee, src, dst, zrows):
    mesh = plsc.VectorSubcoreMesh(core_axis_name="c", subcore_axis_name="s")
    return pl.kernel(
        _edge_body,
        out_type=jax.ShapeDtypeStruct((2, 2, NPAD, ACCW), jnp.float32),
        mesh=mesh,
        scratch_types=[
            pltpu.VMEM((C,), jnp.int32),
            pltpu.VMEM((C,), jnp.int32),
            pltpu.VMEM((C, HD2), jnp.float32),
            pltpu.VMEM((C, HD2), jnp.float32),
            pltpu.VMEM((C, HD2), jnp.float32),
            pltpu.VMEM((C, HD2), jnp.float32),
            pltpu.VMEM((C, ACCW), jnp.float32),
            pltpu.VMEM_SHARED((NPAD, ACCW), jnp.float32),
            pltpu.SemaphoreType.DMA,
        ],
        compiler_params=pltpu.CompilerParams(
            needs_layout_passes=False, use_tc_tiling_on_sc=False),
    )(kh, qh, vh, ee, src, dst, zrows)


def _final_body(p_ref, out):
    lo = p_ref[0, 0] + p_ref[1, 0]
    hi = p_ref[0, 1] + p_ref[1, 1]
    blk = lo.shape[0]
    wv = jnp.concatenate([lo[:, :HD2], hi[:, :HD2]], axis=1)
    z = jnp.concatenate([lo[:, HD2:HD2 + 4], hi[:, HD2:HD2 + 4]], axis=1)
    zb = jnp.broadcast_to(z.reshape(blk, H, 1), (blk, H, DH)).reshape(blk, D)
    out[...] = wv / (zb + 1e-6)


def _final(parts):
    blk = 1024
    grid = (NPAD // blk,)
    return pl.pallas_call(
        _final_body,
        grid=grid,
        in_specs=[pl.BlockSpec((2, 2, blk, ACCW), lambda i: (0, 0, i, 0))],
        out_specs=pl.BlockSpec((blk, D), lambda i: (i, 0)),
        out_shape=jax.ShapeDtypeStruct((NPAD, D), jnp.float32),
    )(parts)


def kernel(x, edge_index, edge_attr, batch_vec, Q_W, Q_b, K_W, K_b,
           E_W, E_b, V_W, V_b):
    qh, kh, vh = _proj(x, Q_W, Q_b.reshape(1, D), K_W, K_b.reshape(1, D),
                       V_W, V_b.reshape(1, D))
    ee = _ee(edge_attr, E_W, E_b.reshape(1, D))
    src = edge_index[0]
    dst = edge_index[1]
    zrows = jnp.zeros((RPS, ACCW), jnp.float32)
    parts = _edge(kh, qh, vh, ee, src, dst, zrows)
    return _final(parts)[:N]
